# baseline (speedup 1.0000x reference)
import functools

import jax
import jax.numpy as jnp
from jax import lax
from jax.experimental import pallas as pl
from jax.experimental.pallas import tpu as pltpu
from jax.experimental.pallas import tpu_sc as plsc

_U = 10000
_I = 10000
_F = 128
_E = 320000
_B = 16384
_LAMADA = 0.001

_NC, _NS, _L = 2, 16, 16
_L2 = 2 * _L
_C = 80
_EPT = _E // _NS
_NCH = _EPT // _C
_SCN = 25
_NSB = _NCH // _SCN
_RB = 80
_NRC = _U // _RB
_BC = 128
_NBC = _B // _NS // _BC

_BLK = 2048

_SC_PARAMS = pltpu.CompilerParams(use_tc_tiling_on_sc=False,
                                  needs_layout_passes=False)
_IL = plsc.PackFormat.INTERLEAVED


def _scale_rows(rows, b, vv, jj):
    @pl.loop(0, _C // _L)
    def _(g):
        vvec = vv[jj, pl.ds(g * _L, _L)]
        for l in range(_L):
            vs = jnp.full((_L,), vvec[l], jnp.float32)
            vb = plsc.pack(vs, vs, format=_IL)
            e = g * _L + l
            for q in range(_F // _L2):
                sl = pl.ds(q * _L2, _L2)
                rows[b, e, sl] = rows[b, e, sl] * vb


def _hop_body(second, *refs):
    if second:
        (gt, em, goff, gsc, vals, dd, wv, gout, pp,
         acc, gi, si, vv, rows, edv, pacc, wvv,
         gsem0, gsem1, gsem2, ssem0, ssem1, ssem2) = refs
    else:
        (gt, goff, gsc, vals, dd, gout,
         acc, gi, si, vv, rows, edv,
         gsem0, gsem1, gsem2, ssem0, ssem1, ssem2) = refs
        em = gt
    gsems = (gsem0, gsem1, gsem2)
    ssems = (ssem0, ssem1, ssem2)

    c = lax.axis_index("c")
    s = lax.axis_index("s")

    if second:
        pltpu.sync_copy(wv, wvv)
        wvec = wvv[...]
        w0, w1, w2 = wvec[0], wvec[1], wvec[2]
        for q in range(4):
            pacc[q, :] = jnp.zeros((_L,), jnp.float32)

    zerov = jnp.zeros((_L2,), jnp.bfloat16)

    @pl.loop(0, _RB)
    def _(r):
        for q in range(_F // _L2):
            rows[0, r, pl.ds(q * _L2, _L2)] = zerov

    @pl.loop(0, (_NRC + _NS - 1) // _NS)
    def _(k):
        ch = s + k * _NS

        @pl.when(ch < _NRC)
        def _():
            pltpu.sync_copy(rows.at[0], acc.at[pl.ds(ch * _RB, _RB), :])

    plsc.subcore_barrier()

    @pl.loop(0, _NSB)
    def _(sb):
        pltpu.sync_copy(goff.at[c, s, pl.ds(sb * _SCN, _SCN)], gi)
        pltpu.sync_copy(gsc.at[c, s, pl.ds(sb * _SCN, _SCN)], si)
        pltpu.sync_copy(vals.at[s, pl.ds(sb * _SCN, _SCN)], vv)
        pltpu.async_copy(gt.at[gi.at[0]], rows.at[0], gsems[0])
        pltpu.async_copy(gt.at[gi.at[1]], rows.at[1], gsems[1])

        @pl.loop(0, _SCN + 2, step=3)
        def _(j):
            for b in range(3):
                jj = j + b
                b1 = (b + 2) % 3

                @pl.when(jj < _SCN)
                def _():
                    pltpu.make_async_copy(gt.at[gi.at[jj]], rows.at[b],
                                          gsems[b]).wait()
                    _scale_rows(rows, b, vv, jj)
                    pltpu.async_copy(rows.at[b], acc.at[si.at[jj]],
                                     ssems[b], add=True)

                    @pl.when(jj + 2 < _SCN)
                    def _():
                        @pl.when(jj >= 1)
                        def _():
                            pltpu.make_async_copy(
                                rows.at[b1], acc.at[si.at[jj - 1]],
                                ssems[b1]).wait()

                        pltpu.async_copy(gt.at[gi.at[jj + 2]], rows.at[b1],
                                         gsems[b1])

        for t in range(3):
            ch = _SCN - 3 + t
            bb = ch % 3
            pltpu.make_async_copy(rows.at[bb], acc.at[si.at[ch]],
                                  ssems[bb]).wait()

    plsc.subcore_barrier()

    @pl.loop(0, (_NRC + _NS - 1) // _NS)
    def _(k):
        ch = s + k * _NS

        @pl.when(ch < _NRC)
        def _():
            r0 = ch * _RB
            pltpu.sync_copy(acc.at[pl.ds(r0, _RB), :], rows.at[0])
            pltpu.sync_copy(gt.at[pl.ds(c * _U + r0, _RB), :], rows.at[1])
            pltpu.sync_copy(dd.at[pl.ds(c * _U + r0, _RB)], edv)
            if second:
                pltpu.sync_copy(em.at[pl.ds(c * _U + r0, _RB), :], rows.at[2])

            @pl.loop(0, _RB // _L)
            def _(gg):
                dvec = edv[pl.ds(gg * _L, _L)]
                for l in range(_L):
                    dv = dvec[l]
                    r = gg * _L + l
                    for q in range(_F // _L2):
                        sl = pl.ds(q * _L2, _L2)
                        a_lo, a_hi = plsc.unpack(rows[0, r, sl], format=_IL)
                        s_lo, s_hi = plsc.unpack(rows[1, r, sl], format=_IL)
                        g_lo = jnp.maximum(a_lo + s_lo * dv, 0.0)
                        g_hi = jnp.maximum(a_hi + s_hi * dv, 0.0)
                        if second:
                            e_lo, e_hi = plsc.unpack(rows[2, r, sl],
                                                     format=_IL)
                            g_lo = e_lo * w0 + s_lo * w1 + g_lo * w2
                            g_hi = e_hi * w0 + s_hi * w1 + g_hi * w2
                            pacc[q, :] = (pacc[q, :] + g_lo * g_lo
                                          + g_hi * g_hi)
                        rows[0, r, sl] = plsc.pack(g_lo, g_hi, format=_IL)

            pltpu.sync_copy(rows.at[0], gout.at[pl.ds(c * _U + r0, _RB), :])

    if second:
        pacc[0, :] = (pacc[0, :] + pacc[1, :]) + (pacc[2, :] + pacc[3, :])
        pltpu.sync_copy(pacc.at[0], pp.at[c * _NS + s])


def _hop1(em, goff, gsc, vals, dd):
    mesh = plsc.VectorSubcoreMesh(core_axis_name="c", subcore_axis_name="s",
                                  num_cores=_NC, num_subcores=_NS)
    return pl.kernel(
        functools.partial(_hop_body, False),
        out_type=jax.ShapeDtypeStruct((2 * _U, _F), jnp.bfloat16),
        mesh=mesh,
        scratch_types=[
            pltpu.VMEM_SHARED((_U, _F), jnp.bfloat16),
            pltpu.VMEM((_SCN, _C), jnp.int32),
            pltpu.VMEM((_SCN, _C), jnp.int32),
            pltpu.VMEM((_SCN, _C), jnp.float32),
            pltpu.VMEM((3, _C, _F), jnp.bfloat16),
            pltpu.VMEM((_RB,), jnp.float32),
            pltpu.SemaphoreType.DMA,
            pltpu.SemaphoreType.DMA,
            pltpu.SemaphoreType.DMA,
            pltpu.SemaphoreType.DMA,
            pltpu.SemaphoreType.DMA,
            pltpu.SemaphoreType.DMA,
        ],
        compiler_params=_SC_PARAMS,
    )(em, goff, gsc, vals, dd)


def _hop2(g1, em, goff, gsc, vals, dd, wv):
    mesh = plsc.VectorSubcoreMesh(core_axis_name="c", subcore_axis_name="s",
                                  num_cores=_NC, num_subcores=_NS)
    return pl.kernel(
        functools.partial(_hop_body, True),
        out_type=(jax.ShapeDtypeStruct((2 * _U, _F), jnp.bfloat16),
                  jax.ShapeDtypeStruct((_NC * _NS, _L), jnp.float32)),
        mesh=mesh,
        scratch_types=[
            pltpu.VMEM_SHARED((_U, _F), jnp.bfloat16),
            pltpu.VMEM((_SCN, _C), jnp.int32),
            pltpu.VMEM((_SCN, _C), jnp.int32),
            pltpu.VMEM((_SCN, _C), jnp.float32),
            pltpu.VMEM((3, _C, _F), jnp.bfloat16),
            pltpu.VMEM((_RB,), jnp.float32),
            pltpu.VMEM((4, _L), jnp.float32),
            pltpu.VMEM((_L,), jnp.float32),
            pltpu.SemaphoreType.DMA,
            pltpu.SemaphoreType.DMA,
            pltpu.SemaphoreType.DMA,
            pltpu.SemaphoreType.DMA,
            pltpu.SemaphoreType.DMA,
            pltpu.SemaphoreType.DMA,
        ],
        compiler_params=_SC_PARAMS,
    )(g1, em, goff, gsc, vals, dd, wv)


def _bgather_body(gt, bidx, out, bgi, brow, sem0, sem1):
    c = lax.axis_index("c")
    s = lax.axis_index("s")
    pltpu.sync_copy(bidx.at[c, s], bgi)
    sems = (sem0, sem1)
    pltpu.async_copy(gt.at[bgi.at[0]], brow.at[0], sem0)

    @pl.loop(0, _NBC, step=2)
    def _(j):
        for b in range(2):
            jj = j + b
            pltpu.make_async_copy(gt.at[bgi.at[jj]], brow.at[b],
                                  sems[b]).wait()

            @pl.when(jj + 1 < _NBC)
            def _():
                pltpu.async_copy(gt.at[bgi.at[jj + 1]], brow.at[1 - b],
                                 sems[1 - b])

            base = c * _B + s * (_NBC * _BC) + jj * _BC
            pltpu.sync_copy(brow.at[b], out.at[pl.ds(base, _BC), :])


def _bgather(gt, bidx):
    mesh = plsc.VectorSubcoreMesh(core_axis_name="c", subcore_axis_name="s",
                                  num_cores=_NC, num_subcores=_NS)
    return pl.kernel(
        _bgather_body,
        out_type=jax.ShapeDtypeStruct((2 * _B, _F), jnp.bfloat16),
        mesh=mesh,
        scratch_types=[
            pltpu.VMEM((_NBC, _BC), jnp.int32),
            pltpu.VMEM((2, _BC, _F), jnp.bfloat16),
            pltpu.SemaphoreType.DMA,
            pltpu.SemaphoreType.DMA,
        ],
        compiler_params=_SC_PARAMS,
    )(gt, bidx)


def _mlp_loss_body(xu_ref, xi_ref, rat_ref, w1_ref, b1_ref, w2_ref, b2_ref,
                   pp_ref, out_ref, sse_ref):
    i = pl.program_id(0)
    nblk = pl.num_programs(0)

    @pl.when(i == 0)
    def _():
        sse_ref[0] = 0.0

    w1 = w1_ref[...]
    w2 = w2_ref[...]
    b1 = b1_ref[...]
    b2 = b2_ref[...]

    def mlp(x):
        h = lax.dot_general(x, w1, (((1,), (1,)), ((), ())),
                            preferred_element_type=jnp.float32) + b1
        h = jnp.where(h > 0, h, 0.1 * h)
        o = lax.dot_general(h, w2, (((1,), (1,)), ((), ())),
                            preferred_element_type=jnp.float32) + b2
        return jnp.where(o > 0, o, 0.1 * o)

    u = mlp(xu_ref[...].astype(jnp.float32))
    v = mlp(xi_ref[...].astype(jnp.float32))
    pred = jnp.sum(u * v, axis=1)
    r = rat_ref[0, 0, :]
    sse_ref[0] += jnp.sum((pred - r) ** 2)

    @pl.when(i == nblk - 1)
    def _():
        loss2 = sse_ref[0] / _B
        reg = _LAMADA * (jnp.sum(pp_ref[0, :]) / (_U * _F)
                         + jnp.sum(pp_ref[1, :]) / (_I * _F))
        cols = lax.broadcasted_iota(jnp.int32, (1, 2), 1)
        out_ref[...] = jnp.where(cols == 0, loss2 + reg, loss2)


def _mlp_loss(xu, xi, ratings, W1, b1, W2, b2, pp):
    nblk = _B // _BLK
    rat2 = ratings.reshape(nblk, 1, _BLK)
    out = pl.pallas_call(
        _mlp_loss_body,
        grid=(nblk,),
        in_specs=[
            pl.BlockSpec((_BLK, _F), lambda i: (i, 0)),
            pl.BlockSpec((_BLK, _F), lambda i: (i, 0)),
            pl.BlockSpec((1, 1, _BLK), lambda i: (i, 0, 0)),
            pl.BlockSpec((2 * _F, _F), lambda i: (0, 0)),
            pl.BlockSpec((1, 2 * _F), lambda i: (0, 0)),
            pl.BlockSpec((_F, 2 * _F), lambda i: (0, 0)),
            pl.BlockSpec((1, _F), lambda i: (0, 0)),
            pl.BlockSpec(pp.shape, lambda i: (0, 0)),
        ],
        out_specs=pl.BlockSpec((1, 2), lambda i: (0, 0)),
        out_shape=jax.ShapeDtypeStruct((1, 2), jnp.float32),
        scratch_shapes=[pltpu.SMEM((1,), jnp.float32)],
    )(xu, xi, rat2, W1, b1.reshape(1, -1), W2, b2.reshape(1, -1), pp)
    return out[0, 0], out[0, 1]


def kernel(user0, item_i0, ratings, u_idx, i_idx, vals, embed_user,
           embed_item, d_i, d_j, W_add, W1, b1, W2, b2):
    em = jnp.concatenate([embed_user, embed_item], axis=0)
    em_bf = em.astype(jnp.bfloat16)
    dd = jnp.concatenate([d_i, d_j], axis=0)
    goff = jnp.stack([i_idx + _U, u_idx]).reshape(_NC, _NS, _NCH, _C)
    gsc = jnp.stack([u_idx, i_idx]).reshape(_NC, _NS, _NCH, _C)
    vals4 = vals.reshape(_NS, _NCH, _C)
    wv = jnp.pad(W_add, (0, _L - 3))
    bidx = jnp.stack([user0, item_i0 + _U]).reshape(_NC, _NS, _NBC, _BC)

    g1 = _hop1(em_bf, goff, gsc, vals4, dd)
    gc, pp = _hop2(g1, em_bf, goff, gsc, vals4, dd, wv)
    xall = _bgather(gc, bidx)
    return _mlp_loss(xall[:_B], xall[_B:], ratings, W1, b1, W2, b2,
                     pp.reshape(2, _NS * _L))


# speedup vs baseline: 1.3233x; 1.3233x over previous
import functools

import jax
import jax.numpy as jnp
from jax import lax
from jax.experimental import pallas as pl
from jax.experimental.pallas import tpu as pltpu
from jax.experimental.pallas import tpu_sc as plsc

_U = 10000
_I = 10000
_F = 128
_E = 320000
_B = 16384
_LAMADA = 0.001

_NC, _NS, _L = 2, 16, 16
_C = 80
_EPT = _E // _NS
_NCH = _EPT // _C
_SCN = 25
_NSB = _NCH // _SCN
_RB = 80
_NRC = _U // _RB
_BC = 64
_NBC = _B // _NS // _BC

_BLK = 2048

_SC_PARAMS = pltpu.CompilerParams(use_tc_tiling_on_sc=False)


def _scale_rows(rows, b, vv, jj):
    @pl.loop(0, _C // _L)
    def _(g):
        vvec = vv[jj, pl.ds(g * _L, _L)]
        for l in range(_L):
            v = vvec[l]
            e = g * _L + l
            for q in range(_F // _L):
                sl = pl.ds(q * _L, _L)
                rows[b, e, sl] = rows[b, e, sl] * v


def _hop_body(second, *refs):
    if second:
        (gt, em, goff, gsc, vals, dd, wv, bidx, gout, pp, xout,
         acc, gi, si, vv, rows, edv, pacc, wvv, bgi,
         gsem0, gsem1, gsem2, ssem0, ssem1, ssem2, stsem) = refs
    else:
        (gt, goff, gsc, vals, dd, gout,
         acc, gi, si, vv, rows, edv,
         gsem0, gsem1, gsem2, ssem0, ssem1, ssem2, stsem) = refs
        em = gt
    gsems = (gsem0, gsem1, gsem2)
    ssems = (ssem0, ssem1, ssem2)

    c = lax.axis_index("c")
    s = lax.axis_index("s")

    if second:
        pltpu.sync_copy(wv, wvv)
        wvec = wvv[...]
        w0, w1, w2 = wvec[0], wvec[1], wvec[2]
        for q in range(_F // _L):
            pacc[q, :] = jnp.zeros((_L,), jnp.float32)

    zero16 = jnp.zeros((_L,), jnp.float32)

    @pl.loop(0, _RB)
    def _(r):
        for q in range(_F // _L):
            rows[0, r, pl.ds(q * _L, _L)] = zero16

    @pl.loop(0, (_NRC + _NS - 1) // _NS)
    def _(k):
        ch = s + k * _NS

        @pl.when(ch < _NRC)
        def _():
            pltpu.sync_copy(rows.at[0], acc.at[pl.ds(ch * _RB, _RB), :])

    plsc.subcore_barrier()

    def _fire_stage(sb, sl):
        pltpu.async_copy(goff.at[c, s, pl.ds(sb * _SCN, _SCN)],
                         gi.at[sl], stsem)
        pltpu.async_copy(gsc.at[c, s, pl.ds(sb * _SCN, _SCN)],
                         si.at[sl], stsem)
        pltpu.async_copy(vals.at[s, pl.ds(sb * _SCN, _SCN)],
                         vv.at[sl], stsem)

    def _wait_stage(sb, sl):
        pltpu.make_async_copy(goff.at[c, s, pl.ds(sb * _SCN, _SCN)],
                              gi.at[sl], stsem).wait()
        pltpu.make_async_copy(gsc.at[c, s, pl.ds(sb * _SCN, _SCN)],
                              si.at[sl], stsem).wait()
        pltpu.make_async_copy(vals.at[s, pl.ds(sb * _SCN, _SCN)],
                              vv.at[sl], stsem).wait()

    _fire_stage(0, 0)

    @pl.loop(0, _NSB, step=2)
    def _(sbo):
        for t in range(2):
            sb = sbo + t
            _wait_stage(sb, t)

            @pl.when(sb + 1 < _NSB)
            def _():
                _fire_stage(sb + 1, 1 - t)

            pltpu.async_copy(gt.at[gi.at[t, 0]], rows.at[0], gsems[0])
            pltpu.async_copy(gt.at[gi.at[t, 1]], rows.at[1], gsems[1])

            @pl.loop(0, _SCN + 2, step=3)
            def _(j):
                for b in range(3):
                    jj = j + b
                    b1 = (b + 2) % 3

                    @pl.when(jj < _SCN)
                    def _():
                        pltpu.make_async_copy(gt.at[gi.at[t, jj]], rows.at[b],
                                              gsems[b]).wait()
                        _scale_rows(rows, b, vv.at[t], jj)
                        pltpu.async_copy(rows.at[b], acc.at[si.at[t, jj]],
                                         ssems[b], add=True)

                        @pl.when(jj + 2 < _SCN)
                        def _():
                            @pl.when(jj >= 1)
                            def _():
                                pltpu.make_async_copy(
                                    rows.at[b1], acc.at[si.at[t, jj - 1]],
                                    ssems[b1]).wait()

                            pltpu.async_copy(gt.at[gi.at[t, jj + 2]],
                                             rows.at[b1], gsems[b1])

            for tt in range(3):
                ch = _SCN - 3 + tt
                bb = ch % 3
                pltpu.make_async_copy(rows.at[bb], acc.at[si.at[t, ch]],
                                      ssems[bb]).wait()

    plsc.subcore_barrier()

    @pl.loop(0, (_NRC + _NS - 1) // _NS)
    def _(k):
        ch = s + k * _NS

        @pl.when(ch < _NRC)
        def _():
            r0 = ch * _RB
            pltpu.sync_copy(acc.at[pl.ds(r0, _RB), :], rows.at[0])
            pltpu.sync_copy(gt.at[pl.ds(c * _U + r0, _RB), :], rows.at[1])
            pltpu.sync_copy(dd.at[pl.ds(c * _U + r0, _RB)], edv)
            if second:
                pltpu.sync_copy(em.at[pl.ds(c * _U + r0, _RB), :], rows.at[2])

            @pl.loop(0, _RB // _L)
            def _(gg):
                dvec = edv[pl.ds(gg * _L, _L)]
                for l in range(_L):
                    dv = dvec[l]
                    r = gg * _L + l
                    for q in range(_F // _L):
                        sl = pl.ds(q * _L, _L)
                        g2 = jnp.maximum(
                            rows[0, r, sl] + rows[1, r, sl] * dv, 0.0)
                        if second:
                            gc = (rows[2, r, sl] * w0 + rows[1, r, sl] * w1
                                  + g2 * w2)
                            pacc[q, :] = pacc[q, :] + gc * gc
                            rows[0, r, sl] = gc
                        else:
                            rows[0, r, sl] = g2

            pltpu.sync_copy(rows.at[0], gout.at[pl.ds(c * _U + r0, _RB), :])

    if second:
        ps = [pacc[q, :] for q in range(_F // _L)]
        pacc[0, :] = (((ps[0] + ps[1]) + (ps[2] + ps[3]))
                      + ((ps[4] + ps[5]) + (ps[6] + ps[7])))
        pltpu.sync_copy(pacc.at[0], pp.at[c * _NS + s])

        plsc.subcore_barrier()
        pltpu.sync_copy(bidx.at[c, s], bgi)
        pltpu.async_copy(gout.at[bgi.at[0]], rows.at[0, pl.ds(0, _BC), :],
                         gsems[0])

        @pl.loop(0, _NBC, step=2)
        def _(j):
            for b in range(2):
                jj = j + b
                pltpu.make_async_copy(gout.at[bgi.at[jj]],
                                      rows.at[b, pl.ds(0, _BC), :],
                                      gsems[b]).wait()

                @pl.when(jj + 1 < _NBC)
                def _():
                    pltpu.async_copy(gout.at[bgi.at[jj + 1]],
                                     rows.at[1 - b, pl.ds(0, _BC), :],
                                     gsems[1 - b])

                base = c * _B + s * (_NBC * _BC) + jj * _BC
                pltpu.sync_copy(rows.at[b, pl.ds(0, _BC), :],
                                xout.at[pl.ds(base, _BC), :])


def _hop1(em, goff, gsc, vals, dd):
    mesh = plsc.VectorSubcoreMesh(core_axis_name="c", subcore_axis_name="s",
                                  num_cores=_NC, num_subcores=_NS)
    return pl.kernel(
        functools.partial(_hop_body, False),
        out_type=jax.ShapeDtypeStruct((2 * _U, _F), jnp.float32),
        mesh=mesh,
        scratch_types=[
            pltpu.VMEM_SHARED((_U, _F), jnp.float32),
            pltpu.VMEM((2, _SCN, _C), jnp.int32),
            pltpu.VMEM((2, _SCN, _C), jnp.int32),
            pltpu.VMEM((2, _SCN, _C), jnp.float32),
            pltpu.VMEM((3, _C, _F), jnp.float32),
            pltpu.VMEM((_RB,), jnp.float32),
            pltpu.SemaphoreType.DMA,
            pltpu.SemaphoreType.DMA,
            pltpu.SemaphoreType.DMA,
            pltpu.SemaphoreType.DMA,
            pltpu.SemaphoreType.DMA,
            pltpu.SemaphoreType.DMA,
            pltpu.SemaphoreType.DMA,
        ],
        compiler_params=_SC_PARAMS,
    )(em, goff, gsc, vals, dd)


def _hop2(g1, em, goff, gsc, vals, dd, wv, bidx):
    mesh = plsc.VectorSubcoreMesh(core_axis_name="c", subcore_axis_name="s",
                                  num_cores=_NC, num_subcores=_NS)
    return pl.kernel(
        functools.partial(_hop_body, True),
        out_type=(jax.ShapeDtypeStruct((2 * _U, _F), jnp.float32),
                  jax.ShapeDtypeStruct((_NC * _NS, _L), jnp.float32),
                  jax.ShapeDtypeStruct((2 * _B, _F), jnp.float32)),
        mesh=mesh,
        scratch_types=[
            pltpu.VMEM_SHARED((_U, _F), jnp.float32),
            pltpu.VMEM((2, _SCN, _C), jnp.int32),
            pltpu.VMEM((2, _SCN, _C), jnp.int32),
            pltpu.VMEM((2, _SCN, _C), jnp.float32),
            pltpu.VMEM((3, _C, _F), jnp.float32),
            pltpu.VMEM((_RB,), jnp.float32),
            pltpu.VMEM((_F // _L, _L), jnp.float32),
            pltpu.VMEM((_L,), jnp.float32),
            pltpu.VMEM((_NBC, _BC), jnp.int32),
            pltpu.SemaphoreType.DMA,
            pltpu.SemaphoreType.DMA,
            pltpu.SemaphoreType.DMA,
            pltpu.SemaphoreType.DMA,
            pltpu.SemaphoreType.DMA,
            pltpu.SemaphoreType.DMA,
            pltpu.SemaphoreType.DMA,
        ],
        compiler_params=_SC_PARAMS,
    )(g1, em, goff, gsc, vals, dd, wv, bidx)


def _mlp_loss_body(xu_ref, xi_ref, rat_ref, w1_ref, b1_ref, w2_ref, b2_ref,
                   pp_ref, out_ref, sse_ref):
    i = pl.program_id(0)
    nblk = pl.num_programs(0)

    @pl.when(i == 0)
    def _():
        sse_ref[0] = 0.0

    w1 = w1_ref[...]
    w2 = w2_ref[...]
    b1 = b1_ref[...]
    b2 = b2_ref[...]

    def mlp(x):
        h = lax.dot_general(x, w1, (((1,), (1,)), ((), ())),
                            preferred_element_type=jnp.float32) + b1
        h = jnp.where(h > 0, h, 0.1 * h)
        o = lax.dot_general(h, w2, (((1,), (1,)), ((), ())),
                            preferred_element_type=jnp.float32) + b2
        return jnp.where(o > 0, o, 0.1 * o)

    u = mlp(xu_ref[...])
    v = mlp(xi_ref[...])
    pred = jnp.sum(u * v, axis=1)
    r = rat_ref[0, 0, :]
    sse_ref[0] += jnp.sum((pred - r) ** 2)

    @pl.when(i == nblk - 1)
    def _():
        loss2 = sse_ref[0] / _B
        reg = _LAMADA * (jnp.sum(pp_ref[0, :]) / (_U * _F)
                         + jnp.sum(pp_ref[1, :]) / (_I * _F))
        cols = lax.broadcasted_iota(jnp.int32, (1, 2), 1)
        out_ref[...] = jnp.where(cols == 0, loss2 + reg, loss2)


def _mlp_loss(xu, xi, ratings, W1, b1, W2, b2, pp):
    nblk = _B // _BLK
    rat2 = ratings.reshape(nblk, 1, _BLK)
    out = pl.pallas_call(
        _mlp_loss_body,
        grid=(nblk,),
        in_specs=[
            pl.BlockSpec((_BLK, _F), lambda i: (i, 0)),
            pl.BlockSpec((_BLK, _F), lambda i: (i, 0)),
            pl.BlockSpec((1, 1, _BLK), lambda i: (i, 0, 0)),
            pl.BlockSpec((2 * _F, _F), lambda i: (0, 0)),
            pl.BlockSpec((1, 2 * _F), lambda i: (0, 0)),
            pl.BlockSpec((_F, 2 * _F), lambda i: (0, 0)),
            pl.BlockSpec((1, _F), lambda i: (0, 0)),
            pl.BlockSpec(pp.shape, lambda i: (0, 0)),
        ],
        out_specs=pl.BlockSpec((1, 2), lambda i: (0, 0)),
        out_shape=jax.ShapeDtypeStruct((1, 2), jnp.float32),
        scratch_shapes=[pltpu.SMEM((1,), jnp.float32)],
    )(xu, xi, rat2, W1, b1.reshape(1, -1), W2, b2.reshape(1, -1), pp)
    return out[0, 0], out[0, 1]


def kernel(user0, item_i0, ratings, u_idx, i_idx, vals, embed_user,
           embed_item, d_i, d_j, W_add, W1, b1, W2, b2):
    em = jnp.concatenate([embed_user, embed_item], axis=0)
    dd = jnp.concatenate([d_i, d_j], axis=0)
    goff = jnp.stack([i_idx + _U, u_idx]).reshape(_NC, _NS, _NCH, _C)
    gsc = jnp.stack([u_idx, i_idx]).reshape(_NC, _NS, _NCH, _C)
    vals4 = vals.reshape(_NS, _NCH, _C)
    wv = jnp.pad(W_add, (0, _L - 3))
    bidx = jnp.stack([user0, item_i0 + _U]).reshape(_NC, _NS, _NBC, _BC)

    g1 = _hop1(em, goff, gsc, vals4, dd)
    gc, pp, xall = _hop2(g1, em, goff, gsc, vals4, dd, wv, bidx)
    del gc
    return _mlp_loss(xall[:_B], xall[_B:], ratings, W1, b1, W2, b2,
                     pp.reshape(2, _NS * _L))
